# Initial kernel scaffold; baseline (speedup 1.0000x reference)
import jax
import jax.numpy as jnp
from jax.experimental import pallas as pl


def kernel(x, edge_index, W_l0, b_l0, W_r0, W_l1, b_l1, W_r1, W_out, b_out):
    raise NotImplementedError("write your pallas kernel here")


# speedup vs baseline: 5.7508x; 5.7508x over previous
import jax
import jax.numpy as jnp
from jax import lax
from jax.experimental import pallas as pl
from jax.experimental.pallas import tpu as pltpu
from jax.experimental.pallas import tpu_sc as plsc

N = 10000
E = 320000
D = 128
NC = 2
NS = 16
NW = NC * NS
B = 128
NB = E // B
FULL_ROUNDS = NB // NW
TAIL = NB - FULL_ROUNDS * NW
RPT = 624
TAIL_ROWS = N - NS * RPT

_mesh = plsc.VectorSubcoreMesh(core_axis_name="c", subcore_axis_name="s")


def _worker_id():
    return lax.axis_index("s") * NC + lax.axis_index("c")


def _n_batches(wid):
    return FULL_ROUNDS + jnp.where(wid < TAIL, 1, 0)


def _sc_agg_body(x_hbm, src_hbm, dst_hbm, out_hbm,
                 src_v, dst_v, rows_v, acc_sh, sem):
    c = lax.axis_index("c")
    s = lax.axis_index("s")
    wid = _worker_id()
    zero16 = jnp.zeros((16,), jnp.float32)

    def init_row(i, carry):
        for j in range(D // 16):
            rows_v[i, pl.ds(j * 16, 16)] = zero16
        return carry

    lax.fori_loop(0, B, init_row, 0)

    base_row = s * RPT
    for k in range(RPT // B):
        pltpu.sync_copy(rows_v, acc_sh.at[pl.ds(base_row + k * B, B)])
    rem = RPT % B
    if rem:
        pltpu.sync_copy(rows_v.at[pl.ds(0, rem)],
                        acc_sh.at[pl.ds(base_row + (RPT // B) * B, rem)])

    @pl.when(s == NS - 1)
    def _zero_tail():
        pltpu.sync_copy(rows_v.at[pl.ds(0, TAIL_ROWS)],
                        acc_sh.at[pl.ds(NS * RPT, TAIL_ROWS)])

    plsc.subcore_barrier()

    def batch_body(b, carry):
        base = (b * NW + wid) * B
        pltpu.sync_copy(src_hbm.at[pl.ds(base, B)], src_v)
        pltpu.sync_copy(dst_hbm.at[pl.ds(base, B)], dst_v)
        pltpu.async_copy(x_hbm.at[src_v], rows_v, sem).wait()
        pltpu.sync_copy(rows_v, acc_sh.at[dst_v], add=True)
        return carry

    lax.fori_loop(0, _n_batches(wid), batch_body, 0)
    plsc.subcore_barrier()

    out_base = c * N + s * RPT
    pltpu.sync_copy(acc_sh.at[pl.ds(s * RPT, RPT)],
                    out_hbm.at[pl.ds(out_base, RPT)])

    @pl.when(s == NS - 1)
    def _write_tail():
        pltpu.sync_copy(acc_sh.at[pl.ds(NS * RPT, TAIL_ROWS)],
                        out_hbm.at[pl.ds(c * N + NS * RPT, TAIL_ROWS)])


_sc_agg = pl.kernel(
    _sc_agg_body,
    out_type=jax.ShapeDtypeStruct((NC * N, D), jnp.float32),
    mesh=_mesh,
    scratch_types=[
        pltpu.VMEM((B,), jnp.int32),
        pltpu.VMEM((B,), jnp.int32),
        pltpu.VMEM((B, D), jnp.float32),
        pltpu.VMEM_SHARED((N, D), jnp.float32),
        pltpu.SemaphoreType.DMA,
    ],
)


def _sc_deg_body(dst_hbm, deg_hbm, dst_v, ones_v, deg_sh, sem):
    c = lax.axis_index("c")
    s = lax.axis_index("s")
    wid = _worker_id()
    del sem
    zero16 = jnp.zeros((16,), jnp.float32)
    one16 = jnp.ones((16,), jnp.float32)

    def zero_row(i, carry):
        for j in range(D // 16):
            ones_v[i, pl.ds(16 * j, 16)] = zero16
        return carry

    lax.fori_loop(0, B, zero_row, 0)

    base_row = s * RPT
    for k in range(RPT // B):
        pltpu.sync_copy(ones_v, deg_sh.at[pl.ds(base_row + k * B, B)])
    rem = RPT % B
    if rem:
        pltpu.sync_copy(ones_v.at[pl.ds(0, rem)],
                        deg_sh.at[pl.ds(base_row + (RPT // B) * B, rem)])

    @pl.when(s == NS - 1)
    def _zero_tail():
        pltpu.sync_copy(ones_v.at[pl.ds(0, TAIL_ROWS)],
                        deg_sh.at[pl.ds(NS * RPT, TAIL_ROWS)])

    def ones_row(i, carry):
        for j in range(D // 16):
            ones_v[i, pl.ds(16 * j, 16)] = one16
        return carry

    lax.fori_loop(0, B, ones_row, 0)
    plsc.subcore_barrier()

    def batch_body(b, carry):
        base = (b * NW + wid) * B
        pltpu.sync_copy(dst_hbm.at[pl.ds(base, B)], dst_v)
        pltpu.sync_copy(ones_v, deg_sh.at[dst_v], add=True)
        return carry

    lax.fori_loop(0, _n_batches(wid), batch_body, 0)
    plsc.subcore_barrier()

    out_base = c * N + s * RPT
    pltpu.sync_copy(deg_sh.at[pl.ds(s * RPT, RPT)],
                    deg_hbm.at[pl.ds(out_base, RPT)])

    @pl.when(s == NS - 1)
    def _write_tail():
        pltpu.sync_copy(deg_sh.at[pl.ds(NS * RPT, TAIL_ROWS)],
                        deg_hbm.at[pl.ds(c * N + NS * RPT, TAIL_ROWS)])


_sc_deg = pl.kernel(
    _sc_deg_body,
    out_type=jax.ShapeDtypeStruct((NC * N, D), jnp.float32),
    mesh=_mesh,
    scratch_types=[
        pltpu.VMEM((B,), jnp.int32),
        pltpu.VMEM((B, D), jnp.float32),
        pltpu.VMEM_SHARED((N, D), jnp.float32),
        pltpu.SemaphoreType.DMA,
    ],
)

BLK = 1000


def _tc_layer_body(p0_r, p1_r, d0_r, d1_r, x_r, wl_r, bl_r, wr_r, o_r):
    agg = p0_r[...] + p1_r[...]
    deg = d0_r[...][:, :1] + d1_r[...][:, :1]
    mean = agg / jnp.maximum(deg, 1.0)
    h = (jnp.dot(mean, wl_r[...], preferred_element_type=jnp.float32)
         + bl_r[...]
         + jnp.dot(x_r[...], wr_r[...], preferred_element_type=jnp.float32))
    o_r[...] = jnp.maximum(h, 0.0)


def _tc_final_body(p0_r, p1_r, d0_r, d1_r, x_r, wl_r, bl_r, wr_r,
                   wo_r, bo_r, o_r):
    agg = p0_r[...] + p1_r[...]
    deg = d0_r[...][:, :1] + d1_r[...][:, :1]
    mean = agg / jnp.maximum(deg, 1.0)
    h = (jnp.dot(mean, wl_r[...], preferred_element_type=jnp.float32)
         + bl_r[...]
         + jnp.dot(x_r[...], wr_r[...], preferred_element_type=jnp.float32))
    h = jnp.maximum(h, 0.0)
    o_r[...] = jnp.dot(h, wo_r[...], preferred_element_type=jnp.float32) + bo_r[...]


_row_spec = pl.BlockSpec((BLK, D), lambda i: (i, 0))
_deg_spec = pl.BlockSpec((BLK, D), lambda i: (i, 0))
_w_spec = pl.BlockSpec((D, D), lambda i: (0, 0))
_b_spec = pl.BlockSpec((1, D), lambda i: (0, 0))

_tc_layer = pl.pallas_call(
    _tc_layer_body,
    grid=(N // BLK,),
    in_specs=[_row_spec, _row_spec, _deg_spec, _deg_spec, _row_spec,
              _w_spec, _b_spec, _w_spec],
    out_specs=_row_spec,
    out_shape=jax.ShapeDtypeStruct((N, D), jnp.float32),
)

_tc_final = pl.pallas_call(
    _tc_final_body,
    grid=(N // BLK,),
    in_specs=[_row_spec, _row_spec, _deg_spec, _deg_spec, _row_spec,
              _w_spec, _b_spec, _w_spec, _w_spec, _b_spec],
    out_specs=_row_spec,
    out_shape=jax.ShapeDtypeStruct((N, D), jnp.float32),
)


def kernel(x, edge_index, W_l0, b_l0, W_r0, W_l1, b_l1, W_r1, W_out, b_out):
    src = edge_index[0].astype(jnp.int32)
    dst = edge_index[1].astype(jnp.int32)
    dg = _sc_deg(dst)
    p = _sc_agg(x, src, dst)
    d0, d1 = dg[:N], dg[N:]
    h = _tc_layer(p[:N], p[N:], d0, d1, x,
                  W_l0, b_l0.reshape(1, D), W_r0)
    q = _sc_agg(h, src, dst)
    out = _tc_final(q[:N], q[N:], d0, d1, h,
                    W_l1, b_l1.reshape(1, D), W_r1,
                    W_out, b_out.reshape(1, D))
    return out
